# baseline (speedup 1.0000x reference)
import functools

import jax
import jax.numpy as jnp
import numpy as np
from jax import lax
from jax.experimental import pallas as pl


def _seg_matrix(rows: int, cols: int, group: int, transpose: bool):
    if transpose:
        ll = lax.broadcasted_iota(jnp.int32, (rows, cols), 0)
        kk = lax.broadcasted_iota(jnp.int32, (rows, cols), 1)
    else:
        kk = lax.broadcasted_iota(jnp.int32, (rows, cols), 0)
        ll = lax.broadcasted_iota(jnp.int32, (rows, cols), 1)
    sel = (kk >= ll * group) & (kk < ll * group + group)
    return jnp.where(sel, jnp.float32(1), jnp.float32(0)).astype(jnp.bfloat16)


def _strip_body(x_ref, o_ref, *, rows, qblk, group, target, lw, hw):
    xv = x_ref[...]
    x2 = xv * xv
    a16 = x2.astype(jnp.bfloat16)
    b16 = (x2 - a16.astype(jnp.float32)).astype(jnp.bfloat16)
    cmat = _seg_matrix(qblk * group, qblk, group, transpose=False)
    n2 = (jnp.dot(a16, cmat, preferred_element_type=jnp.float32)
          + jnp.dot(b16, cmat, preferred_element_type=jnp.float32))

    bits = lax.bitcast_convert_type(n2, jnp.int32)

    def step(i, t):
        cand = t | lax.shift_left(jnp.int32(1), jnp.int32(30) - i)
        cnt = jnp.sum((bits >= cand).astype(jnp.int32), axis=0, keepdims=True)
        return jnp.where(cnt >= target, cand, t)

    tbits = lax.fori_loop(0, 31, step, jnp.zeros((1, qblk), jnp.int32))
    cnt_t = jnp.sum((bits >= tbits).astype(jnp.int32), axis=0, keepdims=True)
    v_hi = lax.bitcast_convert_type(tbits, jnp.float32)
    below = jnp.max(jnp.where(bits < tbits, bits, 0), axis=0, keepdims=True)
    v_lo = jnp.where(cnt_t >= target + 1, v_hi,
                     lax.bitcast_convert_type(below, jnp.float32))
    thres = jnp.sqrt(v_lo) * lw + jnp.sqrt(v_hi) * hw

    keep = (jnp.sqrt(n2) >= thres).astype(jnp.float32).astype(jnp.bfloat16)
    emat = _seg_matrix(qblk, qblk * group, group, transpose=True)
    mask9 = jnp.dot(keep, emat, preferred_element_type=jnp.float32)
    o_ref[...] = xv * mask9


@jax.jit
def kernel(x):
    p, q, k1, k2 = x.shape
    group = k1 * k2
    qblk = 128
    grid = q // qblk
    xf = x.reshape(p, q * group)

    qs = np.float32(0.9) * np.float32(p - 1)
    low = np.floor(qs)
    hw = np.float32(qs - low)
    lw = np.float32(np.float32(1.0) - hw)
    target = int(p - np.ceil(qs))

    body = functools.partial(_strip_body, rows=p, qblk=qblk, group=group,
                             target=target, lw=lw, hw=hw)
    out = pl.pallas_call(
        body,
        grid=(grid,),
        in_specs=[pl.BlockSpec((p, qblk * group), lambda i: (0, i))],
        out_specs=pl.BlockSpec((p, qblk * group), lambda i: (0, i)),
        out_shape=jax.ShapeDtypeStruct((p, q * group), jnp.float32),
    )(xf)
    return out.reshape(p, q, k1, k2)


# speedup vs baseline: 6.0274x; 6.0274x over previous
import functools

import jax
import jax.numpy as jnp
import numpy as np
from jax import lax
from jax.experimental import pallas as pl


def _strip_body(x_ref, o_ref, *, target, lw, hw):
    xv = x_ref[...]
    n2 = jnp.sum(xv * xv, axis=0)
    qblk = n2.shape[1]

    bits = lax.bitcast_convert_type(n2, jnp.int32)

    def step(i, t):
        cand = t | lax.shift_left(jnp.int32(1), jnp.int32(30) - i)
        cnt = jnp.sum((bits >= cand).astype(jnp.int32), axis=0, keepdims=True)
        return jnp.where(cnt >= target, cand, t)

    tbits = lax.fori_loop(0, 31, step, jnp.zeros((1, qblk), jnp.int32))
    cnt_t = jnp.sum((bits >= tbits).astype(jnp.int32), axis=0, keepdims=True)
    v_hi = lax.bitcast_convert_type(tbits, jnp.float32)
    below = jnp.max(jnp.where(bits < tbits, bits, 0), axis=0, keepdims=True)
    v_lo = jnp.where(cnt_t >= target + 1, v_hi,
                     lax.bitcast_convert_type(below, jnp.float32))
    thres = jnp.sqrt(v_lo) * lw + jnp.sqrt(v_hi) * hw

    keep = (jnp.sqrt(n2) >= thres).astype(jnp.float32)
    o_ref[...] = xv * keep[None, :, :]


@jax.jit
def kernel(x):
    p, q, k1, k2 = x.shape
    group = k1 * k2
    qblk = 128
    grid = q // qblk
    xt = x.transpose(2, 3, 0, 1).reshape(group, p, q)

    qs = np.float32(0.9) * np.float32(p - 1)
    low = np.floor(qs)
    hw = np.float32(qs - low)
    lw = np.float32(np.float32(1.0) - hw)
    target = int(p - np.ceil(qs))

    body = functools.partial(_strip_body, target=target, lw=lw, hw=hw)
    out = pl.pallas_call(
        body,
        grid=(grid,),
        in_specs=[pl.BlockSpec((group, p, qblk), lambda i: (0, 0, i))],
        out_specs=pl.BlockSpec((group, p, qblk), lambda i: (0, 0, i)),
        out_shape=jax.ShapeDtypeStruct((group, p, q), jnp.float32),
    )(xt)
    return out.reshape(k1, k2, p, q).transpose(2, 3, 0, 1)


# speedup vs baseline: 6.4428x; 1.0689x over previous
import functools

import jax
import jax.numpy as jnp
import numpy as np
from jax import lax
from jax.experimental import pallas as pl


def _strip_body(x_ref, o_ref, *, target, lw, hw):
    group = x_ref.shape[0]
    n2 = x_ref[0] * x_ref[0]
    for r in range(1, group):
        plane = x_ref[r]
        n2 = n2 + plane * plane
    qblk = n2.shape[1]

    bits = lax.bitcast_convert_type(n2, jnp.int32)

    tbits = jnp.zeros((1, qblk), jnp.int32)
    for b in range(30, -1, -1):
        cand = tbits | jnp.int32(1 << b)
        cnt = jnp.sum((bits >= cand).astype(jnp.int32), axis=0, keepdims=True)
        tbits = jnp.where(cnt >= target, cand, tbits)
    cnt_t = jnp.sum((bits >= tbits).astype(jnp.int32), axis=0, keepdims=True)
    v_hi = lax.bitcast_convert_type(tbits, jnp.float32)
    below = jnp.max(jnp.where(bits < tbits, bits, 0), axis=0, keepdims=True)
    v_lo = jnp.where(cnt_t >= target + 1, v_hi,
                     lax.bitcast_convert_type(below, jnp.float32))
    thres = jnp.sqrt(v_lo) * lw + jnp.sqrt(v_hi) * hw

    keep = (jnp.sqrt(n2) >= thres).astype(jnp.float32)
    for r in range(group):
        o_ref[r] = x_ref[r] * keep


@jax.jit
def kernel(x):
    p, q, k1, k2 = x.shape
    group = k1 * k2
    qblk = 128
    grid = q // qblk
    xt = x.transpose(2, 3, 0, 1).reshape(group, p, q)

    qs = np.float32(0.9) * np.float32(p - 1)
    low = np.floor(qs)
    hw = np.float32(qs - low)
    lw = np.float32(np.float32(1.0) - hw)
    target = int(p - np.ceil(qs))

    body = functools.partial(_strip_body, target=target, lw=lw, hw=hw)
    out = pl.pallas_call(
        body,
        grid=(grid,),
        in_specs=[pl.BlockSpec((group, p, qblk), lambda i: (0, 0, i))],
        out_specs=pl.BlockSpec((group, p, qblk), lambda i: (0, 0, i)),
        out_shape=jax.ShapeDtypeStruct((group, p, q), jnp.float32),
    )(xt)
    return out.reshape(k1, k2, p, q).transpose(2, 3, 0, 1)
